# Initial kernel scaffold; baseline (speedup 1.0000x reference)
import jax
import jax.numpy as jnp
from jax.experimental import pallas as pl


def kernel(tokens, tables):
    raise NotImplementedError("write your pallas kernel here")


# speedup vs baseline: 1.1977x; 1.1977x over previous
import jax
import jax.numpy as jnp
from jax import lax
from jax.experimental import pallas as pl
from jax.experimental.pallas import tpu as pltpu
from jax.experimental.pallas import tpu_sc as plsc

_NUM_FIELDS = 26
_VOCAB = 100000
_EMBED_DIM = 32
_BATCH = 16384
_ROWS = _BATCH * _NUM_FIELDS
_NC, _NS, _L = 2, 16, 16
_NW = _NC * _NS
_RPW = _ROWS // _NW
_IDX_MINOR = 128
_IDX_ROWS = _RPW // _IDX_MINOR
_SUBS = 4
_CHUNK = _SUBS * _IDX_MINOR
_NMACRO = _RPW // _CHUNK


def _body(tok_hbm, tab_hbm, out_hbm, idx_v, rows_v, gsem):
    wid = lax.axis_index("s") * _NC + lax.axis_index("c")
    row_base = wid * _RPW
    pltpu.sync_copy(tok_hbm.at[pl.ds(wid * _IDX_ROWS, _IDX_ROWS)], idx_v)
    lanes = lax.iota(jnp.int32, _L)

    def macro(m, carry):
        for g in range(_SUBS):
            j = m * _SUBS + g
            for k in range(_IDX_MINOR // _L):
                s = k * _L
                r0 = row_base + j * _IDX_MINOR + s
                fld = lax.rem(lanes + r0, _NUM_FIELDS)
                idx_v[j, pl.ds(s, _L)] = idx_v[j, pl.ds(s, _L)] + fld * _VOCAB
        cps = []
        for g in range(_SUBS):
            j = m * _SUBS + g
            cps.append(pltpu.async_copy(
                tab_hbm.at[idx_v.at[j]],
                rows_v.at[pl.ds(g * _IDX_MINOR, _IDX_MINOR)],
                gsem))
        for cp in cps:
            cp.wait()
        pltpu.sync_copy(rows_v, out_hbm.at[pl.ds(row_base + m * _CHUNK, _CHUNK)])
        return carry

    lax.fori_loop(0, _NMACRO, macro, 0)


def kernel(tokens, tables):
    tok = tokens.reshape(_ROWS // _IDX_MINOR, _IDX_MINOR).astype(jnp.int32)
    tab = tables.reshape(_NUM_FIELDS * _VOCAB, _EMBED_DIM)
    mesh = plsc.VectorSubcoreMesh(core_axis_name="c", subcore_axis_name="s")
    run = pl.kernel(
        _body,
        mesh=mesh,
        out_type=jax.ShapeDtypeStruct((_ROWS, _EMBED_DIM), jnp.float32),
        scratch_types=[
            pltpu.VMEM((_IDX_ROWS, _IDX_MINOR), jnp.int32),
            pltpu.VMEM((_CHUNK, _EMBED_DIM), jnp.float32),
            pltpu.SemaphoreType.DMA,
        ],
        compiler_params=pltpu.CompilerParams(use_tc_tiling_on_sc=False),
    )
    out = run(tok, tab)
    return out.reshape(_BATCH, _NUM_FIELDS * _EMBED_DIM)


# speedup vs baseline: 1.3678x; 1.1421x over previous
import jax
import jax.numpy as jnp
from jax import lax
from jax.experimental import pallas as pl
from jax.experimental.pallas import tpu as pltpu
from jax.experimental.pallas import tpu_sc as plsc

_NUM_FIELDS = 26
_VOCAB = 100000
_EMBED_DIM = 32
_BATCH = 16384
_NC, _NS, _L = 2, 16, 16
_NW = _NC * _NS
_BPW = _BATCH // _NW
_IDX_MINOR = 128
_GPF = _BPW // _IDX_MINOR


def _body(tok_hbm, tab_hbm, out_hbm, tokv, idxv, rows, gsem, ssem0, ssem1):
    wid = lax.axis_index("s") * _NC + lax.axis_index("c")
    b0 = wid * _BPW
    pltpu.sync_copy(tok_hbm.at[pl.ds(b0, _BPW)], tokv)
    lanes = lax.iota(jnp.int32, _L)
    ssems = (ssem0, ssem1)

    def extract(i):
        for g in range(_GPF):
            for h in range(_IDX_MINOR // _L):
                rows_id = lanes + (g * _IDX_MINOR + h * _L)
                cols_id = jnp.full((_L,), 0, jnp.int32) + i
                idxv[g, pl.ds(h * _L, _L)] = plsc.load_gather(
                    tokv, [rows_id, cols_id])

    def field(i, slot):
        extract(i)
        cps = []
        for g in range(_GPF):
            cps.append(pltpu.async_copy(
                tab_hbm.at[i].at[idxv.at[g]],
                rows.at[slot, pl.ds(g * _IDX_MINOR, _IDX_MINOR)],
                gsem))
        for cp in cps:
            cp.wait()
        pltpu.async_copy(
            rows.at[slot],
            out_hbm.at[pl.ds(b0, _BPW), pl.ds(i * _EMBED_DIM, _EMBED_DIM)],
            ssems[slot])

    def pair(p, carry):
        i = p * 2
        for b in range(2):
            @pl.when(p > 0)
            def _():
                pltpu.make_async_copy(
                    rows.at[b],
                    out_hbm.at[pl.ds(b0, _BPW), pl.ds(0, _EMBED_DIM)],
                    ssems[b]).wait()
            field(i + b, b)
        return carry

    lax.fori_loop(0, _NUM_FIELDS // 2, pair, 0)
    for b in range(2):
        pltpu.make_async_copy(
            rows.at[b],
            out_hbm.at[pl.ds(b0, _BPW), pl.ds(0, _EMBED_DIM)],
            ssems[b]).wait()


def kernel(tokens, tables):
    tok = tokens.astype(jnp.int32)
    mesh = plsc.VectorSubcoreMesh(core_axis_name="c", subcore_axis_name="s")
    run = pl.kernel(
        _body,
        mesh=mesh,
        out_type=jax.ShapeDtypeStruct(
            (_BATCH, _NUM_FIELDS * _EMBED_DIM), jnp.float32),
        scratch_types=[
            pltpu.VMEM((_BPW, _NUM_FIELDS), jnp.int32),
            pltpu.VMEM((_GPF, _IDX_MINOR), jnp.int32),
            pltpu.VMEM((2, _BPW, _EMBED_DIM), jnp.float32),
            pltpu.SemaphoreType.DMA,
            pltpu.SemaphoreType.DMA,
            pltpu.SemaphoreType.DMA,
        ],
        compiler_params=pltpu.CompilerParams(
            use_tc_tiling_on_sc=False, needs_layout_passes=False),
    )
    return run(tok, tables)


# speedup vs baseline: 1.8380x; 1.3437x over previous
import jax
import jax.numpy as jnp
from jax import lax
from jax.experimental import pallas as pl
from jax.experimental.pallas import tpu as pltpu
from jax.experimental.pallas import tpu_sc as plsc

_NUM_FIELDS = 26
_VOCAB = 100000
_EMBED_DIM = 32
_BATCH = 16384
_NC, _NS, _L = 2, 16, 16
_NW = _NC * _NS
_CHUNK = 2048
_NCH = _BATCH // _CHUNK
_GRP = _CHUNK // _L


def _body(tok_hbm, tab_hbm, out_hbm, tokv, rowv, outv, ssem0, ssem1):
    d = lax.axis_index("s") * _NC + lax.axis_index("c")
    ssems = (ssem0, ssem1)

    def task(i, carry):
        r = i * _EMBED_DIM + d
        pltpu.sync_copy(tok_hbm.at[i], tokv)
        pltpu.sync_copy(tab_hbm.at[i, d], rowv)
        for c in range(_NCH):
            slot = c & 1

            def wait_slot(slot=slot):
                pltpu.make_async_copy(
                    outv.at[slot], out_hbm.at[r, pl.ds(0, _CHUNK)],
                    ssems[slot]).wait()

            if c >= 2:
                wait_slot()
            else:
                pl.when(i > 0)(wait_slot)

            def grp(g, _, c=c, slot=slot):
                idx = tokv[pl.ds(c * _CHUNK + g * _L, _L)]
                outv[slot, pl.ds(g * _L, _L)] = plsc.load_gather(rowv, [idx])
                return _

            lax.fori_loop(0, _GRP, grp, 0)
            pltpu.async_copy(
                outv.at[slot], out_hbm.at[r, pl.ds(c * _CHUNK, _CHUNK)],
                ssems[slot])
        return carry

    lax.fori_loop(0, _NUM_FIELDS, task, 0)
    for slot in range(2):
        pltpu.make_async_copy(
            outv.at[slot], out_hbm.at[0, pl.ds(0, _CHUNK)],
            ssems[slot]).wait()


def kernel(tokens, tables):
    tok = tokens.T.astype(jnp.int32)
    tab = tables.transpose(0, 2, 1)
    mesh = plsc.VectorSubcoreMesh(core_axis_name="c", subcore_axis_name="s")
    run = pl.kernel(
        _body,
        mesh=mesh,
        out_type=jax.ShapeDtypeStruct(
            (_NUM_FIELDS * _EMBED_DIM, _BATCH), jnp.float32),
        scratch_types=[
            pltpu.VMEM((_BATCH,), jnp.int32),
            pltpu.VMEM((_VOCAB,), jnp.float32),
            pltpu.VMEM((2, _CHUNK), jnp.float32),
            pltpu.SemaphoreType.DMA,
            pltpu.SemaphoreType.DMA,
        ],
        compiler_params=pltpu.CompilerParams(
            use_tc_tiling_on_sc=False, needs_layout_passes=False),
    )
    out_t = run(tok, tab)
    return out_t.T


# speedup vs baseline: 3.8112x; 2.0736x over previous
import jax
import jax.numpy as jnp
from jax import lax
from jax.experimental import pallas as pl
from jax.experimental.pallas import tpu as pltpu
from jax.experimental.pallas import tpu_sc as plsc

_NUM_FIELDS = 26
_VOCAB = 100000
_EMBED_DIM = 32
_BATCH = 16384
_NC, _NS, _L = 2, 16, 16
_NW = _NC * _NS
_CHUNK = 2048
_NCH = _BATCH // _CHUNK
_GRP = _CHUNK // _L


def _body(tok_hbm, tab_hbm, out_hbm, tokv, rowv, outv, ssem0, ssem1):
    d = lax.axis_index("s") * _NC + lax.axis_index("c")
    ssems = (ssem0, ssem1)

    def task(i, carry):
        r = i * _EMBED_DIM + d
        pltpu.sync_copy(tok_hbm.at[i], tokv)
        pltpu.sync_copy(tab_hbm.at[i, d], rowv)
        for c in range(_NCH):
            slot = c & 1

            def wait_slot(slot=slot):
                pltpu.make_async_copy(
                    outv.at[slot], out_hbm.at[r, pl.ds(0, _CHUNK)],
                    ssems[slot]).wait()

            if c >= 2:
                wait_slot()
            else:
                pl.when(i > 0)(wait_slot)

            def grp(g, _, c=c, slot=slot):
                idx = tokv[pl.ds(c * _CHUNK + g * _L, _L)]
                outv[slot, pl.ds(g * _L, _L)] = plsc.load_gather(rowv, [idx])
                return _

            lax.fori_loop(0, _GRP, grp, 0)
            pltpu.async_copy(
                outv.at[slot], out_hbm.at[r, pl.ds(c * _CHUNK, _CHUNK)],
                ssems[slot])
        return carry

    lax.fori_loop(0, _NUM_FIELDS, task, 0)
    for slot in range(2):
        pltpu.make_async_copy(
            outv.at[slot], out_hbm.at[0, pl.ds(0, _CHUNK)],
            ssems[slot]).wait()


def kernel(tokens, tables):
    tok = tokens.T.astype(jnp.int32)
    tab = tables.transpose(0, 2, 1)
    mesh = plsc.VectorSubcoreMesh(core_axis_name="c", subcore_axis_name="s")
    run = pl.kernel(
        _body,
        mesh=mesh,
        out_type=jax.ShapeDtypeStruct(
            (_NUM_FIELDS * _EMBED_DIM, _BATCH), jnp.float32),
        scratch_types=[
            pltpu.VMEM((_BATCH,), jnp.int32),
            pltpu.VMEM((_VOCAB,), jnp.float32),
            pltpu.VMEM((2, _CHUNK), jnp.float32),
            pltpu.SemaphoreType.DMA,
            pltpu.SemaphoreType.DMA,
        ],
        compiler_params=pltpu.CompilerParams(
            use_tc_tiling_on_sc=True, needs_layout_passes=False),
    )
    out_t = run(tok, tab)
    return out_t.T
